# baseline (speedup 1.0000x reference)
import functools

import jax
import jax.numpy as jnp
from jax import lax
from jax.experimental import pallas as pl
from jax.experimental.pallas import tpu as pltpu
from jax.experimental.pallas import tpu_sc as plsc

N = 10000
NH = N // 2
D = 256
E = 160000
NW = 32
EPW = E // NW
B = 2000
CAP = E + 4 * B + 16
DUMP = 16 * NH
RB = 1000

_mesh = lambda: plsc.VectorSubcoreMesh(
    core_axis_name="c", subcore_axis_name="s", num_cores=2, num_subcores=16)
_sc_params = pltpu.CompilerParams(
    needs_layout_passes=False, use_tc_tiling_on_sc=False)


def _rup_t(x):
    return ((x + 2 * B - 1) // (2 * B)) * (2 * B)


@functools.partial(
    pl.kernel,
    out_type=(
        jax.ShapeDtypeStruct((NW, N), jnp.float32),
        jax.ShapeDtypeStruct((NW * 16,), jnp.int32),
    ),
    mesh=_mesh(),
    scratch_types=[
        pltpu.VMEM((EPW + 16,), jnp.int32),
        pltpu.VMEM((16,), jnp.int32),
        pltpu.VMEM((8 * N,), jnp.float32),
    ],
    compiler_params=_sc_params,
)
def _deg_kernel(dst_hbm, degp_hbm, cnt_hbm, dstb, cntb, acc):
    wid = lax.axis_index("s") * 2 + lax.axis_index("c")
    base = wid * EPW
    pltpu.sync_copy(dst_hbm.at[pl.ds(base, EPW)], dstb.at[pl.ds(0, EPW)])

    lanes = lax.iota(jnp.int32, 16)
    offs = (lanes & 7) * N
    mlo = lanes < 8
    mhi = lanes >= 8
    ones = jnp.ones((16,), jnp.float32)
    zero = jnp.zeros((16,), jnp.float32)

    @plsc.parallel_loop(0, (8 * N) // 16, unroll=8)
    def _(i):
        acc[pl.ds(i * 16, 16)] = zero
    cntb[pl.ds(0, 16)] = jnp.zeros((16,), jnp.int32)

    nfull = EPW // 16

    @plsc.parallel_loop(0, nfull, unroll=8)
    def _(i):
        dv = dstb[pl.ds(i * 16, 16)]
        plsc.addupdate(cntb.at[pl.ds(0, 16)], jnp.where(dv < NH, 1, 0))
        addr = dv + offs
        plsc.addupdate_scatter(acc, [addr], ones, mask=mlo)
        plsc.addupdate_scatter(acc, [addr], ones, mask=mhi)

    dv = dstb[pl.ds(nfull * 16, 16)]
    plsc.addupdate(cntb.at[pl.ds(0, 16)], jnp.where((dv < NH) & mlo, 1, 0))
    plsc.addupdate_scatter(acc, [dv + offs], ones, mask=mlo)

    @plsc.parallel_loop(0, N // 16, unroll=4)
    def _(i):
        s = acc[pl.ds(i * 16, 16)]
        for k in range(1, 8):
            s = s + acc[pl.ds(k * N + i * 16, 16)]
        acc[pl.ds(i * 16, 16)] = s

    pltpu.sync_copy(cntb, cnt_hbm.at[pl.ds(wid * 16, 16)])
    pltpu.sync_copy(acc.at[pl.ds(0, N)], degp_hbm.at[wid])


@functools.partial(
    pl.kernel,
    out_type=(
        jax.ShapeDtypeStruct((CAP,), jnp.int32),
        jax.ShapeDtypeStruct((CAP,), jnp.int32),
    ),
    mesh=_mesh(),
    scratch_types=[
        pltpu.VMEM((EPW + 16,), jnp.int32),
        pltpu.VMEM((EPW + 16,), jnp.int32),
        pltpu.VMEM((NW * 16,), jnp.int32),
        pltpu.VMEM((EPW + 32,), jnp.int32),
        pltpu.VMEM((EPW + 32,), jnp.int32),
        pltpu.VMEM((EPW + 32,), jnp.int32),
        pltpu.VMEM((EPW + 32,), jnp.int32),
        pltpu.VMEM((32,), jnp.int32),
        pltpu.VMEM((32,), jnp.int32),
        pltpu.SemaphoreType.DMA,
        pltpu.SemaphoreType.DMA,
    ],
    compiler_params=_sc_params,
)
def _part_kernel(src_hbm, dst_hbm, cnt_hbm, srcp_hbm, dstp_hbm,
                 srcb, dstb, cntb, ls, ld, hs, hd, pv, vv, sem, sem2):
    wid = lax.axis_index("s") * 2 + lax.axis_index("c")
    base = wid * EPW
    pltpu.sync_copy(src_hbm.at[pl.ds(base, EPW)], srcb.at[pl.ds(0, EPW)])
    pltpu.sync_copy(dst_hbm.at[pl.ds(base, EPW)], dstb.at[pl.ds(0, EPW)])
    pltpu.sync_copy(cnt_hbm, cntb)

    lanes = lax.iota(jnp.int32, 16)
    mlo8 = lanes < 8

    def csum_rows(hi, init):
        def sb(v, s):
            return s + cntb[pl.ds(v * 16, 16)]
        return jnp.sum(lax.fori_loop(0, hi, sb, init))

    zero16 = jnp.zeros((16,), jnp.int32)
    e0 = csum_rows(NW, zero16)
    base_lo = csum_rows(wid, zero16)
    o1 = _rup_t(e0)
    base_hi = o1 + (wid * EPW - base_lo)

    olo0 = base_lo % 16
    ohi0 = base_hi % 16
    nfull = EPW // 16

    def pb(i, carry):
        olo, ohi = carry
        j = i * 16
        valid = jnp.where(i < nfull, jnp.full((16,), 1, jnp.int32),
                          jnp.where(mlo8, 1, 0))
        sv = srcb[pl.ds(j, 16)]
        dv = dstb[pl.ds(j, 16)]
        m = dv < NH
        milo = jnp.where(m, 1, 0) * valid
        mihi = (1 - jnp.where(m, 1, 0)) * valid
        clo = plsc.cumsum(milo)
        chi = plsc.cumsum(mihi)
        plo = olo + clo - 1
        phi = ohi + chi - 1
        mlo_b = milo > 0
        mhi_b = mihi > 0
        s16 = sv * 16
        plsc.store_scatter(ls, [plo], s16, mask=mlo_b)
        plsc.store_scatter(ld, [plo], dv * 16, mask=mlo_b)
        plsc.store_scatter(hs, [phi], s16, mask=mhi_b)
        plsc.store_scatter(hd, [phi], (dv - NH) * 16, mask=mhi_b)
        return olo + jnp.sum(milo), ohi + jnp.sum(mihi)
    oloF, ohiF = lax.fori_loop(0, nfull + 1, pb, (olo0, ohi0))
    cw = oloF - olo0
    chw = ohiF - ohi0

    def writeout(buf, gbase, c, out_hbm):
        sb = gbase % 16
        hl = jnp.minimum((16 - sb) % 16, c)
        rem = c - hl
        nch = rem // 512
        n16 = (rem - nch * 512) // 16
        tl = rem - nch * 512 - n16 * 16
        ab = pl.multiple_of(sb + hl, 16)
        gab = pl.multiple_of(gbase + hl, 16)

        def big(i, _):
            pltpu.async_copy(buf.at[pl.ds(ab + i * 512, 512)],
                             out_hbm.at[pl.ds(gab + i * 512, 512)], sem)
            return 0
        lax.fori_loop(0, nch, big, 0)

        def med(i, _):
            pltpu.async_copy(buf.at[pl.ds(ab + nch * 512 + i * 16, 16)],
                             out_hbm.at[pl.ds(gab + nch * 512 + i * 16, 16)],
                             sem)
            return 0
        lax.fori_loop(0, n16, med, 0)

        tb = pl.multiple_of(ab + nch * 512 + n16 * 16, 16)
        pv[pl.ds(0, 16)] = jnp.where(
            (lanes >= sb) & (lanes < sb + hl), gbase - sb + lanes,
            CAP - 16 + lanes)
        vv[pl.ds(0, 16)] = buf[pl.ds(0, 16)]
        pv[pl.ds(16, 16)] = jnp.where(
            lanes < tl, gbase + (tb - sb) + lanes, CAP - 16 + lanes)
        vv[pl.ds(16, 16)] = buf[pl.ds(tb, 16)]
        pltpu.async_copy(vv, out_hbm.at[pv], sem2).wait()

        def bigw(i, _):
            pltpu.make_async_copy(buf.at[pl.ds(ab + i * 512, 512)],
                                  out_hbm.at[pl.ds(gab + i * 512, 512)],
                                  sem).wait()
            return 0
        lax.fori_loop(0, nch, bigw, 0)

        def medw(i, _):
            pltpu.make_async_copy(
                buf.at[pl.ds(ab + nch * 512 + i * 16, 16)],
                out_hbm.at[pl.ds(gab + nch * 512 + i * 16, 16)], sem).wait()
            return 0
        lax.fori_loop(0, n16, medw, 0)

    writeout(ls, base_lo, cw, srcp_hbm)
    writeout(ld, base_lo, cw, dstp_hbm)
    writeout(hs, base_hi, chw, srcp_hbm)
    writeout(hd, base_hi, chw, dstp_hbm)

    @pl.when(wid == NW - 1)
    def _():
        e1 = E - e0
        s1end = o1 + _rup_t(e1)

        @plsc.parallel_loop(0, (2 * B + 32) // 16, unroll=4)
        def _(i):
            ls[pl.ds(i * 16, 16)] = jnp.zeros((16,), jnp.int32)
            ld[pl.ds(i * 16, 16)] = jnp.full((16,), DUMP, jnp.int32)
        writeout(ls, e0, o1 - e0, srcp_hbm)
        writeout(ld, e0, o1 - e0, dstp_hbm)
        writeout(ls, o1 + e1, s1end - o1 - e1, srcp_hbm)
        writeout(ld, o1 + e1, s1end - o1 - e1, dstp_hbm)


def _make_agg(npass):
    nv16 = N * 16 - 15

    @functools.partial(
        pl.kernel,
        out_type=jax.ShapeDtypeStruct((16 * npass, 16 * N), jnp.float32),
        mesh=_mesh(),
        scratch_types=[
            pltpu.VMEM((2, B), jnp.int32),
            pltpu.VMEM((2, B), jnp.int32),
            pltpu.VMEM((2, B, 8), jnp.int32),
            pltpu.VMEM((16 * NH + 16,), jnp.float32),
            pltpu.VMEM((NW * 16,), jnp.int32),
            pltpu.SemaphoreType.DMA,
            pltpu.SemaphoreType.DMA,
            pltpu.SemaphoreType.DMA,
            pltpu.SemaphoreType.DMA,
            pltpu.SemaphoreType.DMA,
            pltpu.SemaphoreType.DMA,
        ],
        compiler_params=_sc_params,
    )
    def agg(x3_hbm, srcp_hbm, dstp_hbm, cnt_hbm, out_hbm,
            idxb, dstb, rows, acc, cntb, s0, s1, g0, g1, d0, d1):
        cid = lax.axis_index("c")
        sid = lax.axis_index("s")
        lanes = lax.iota(jnp.int32, 16)
        zero = jnp.zeros((16,), jnp.float32)

        pltpu.sync_copy(cnt_hbm, cntb)

        def sb(v, s):
            return s + cntb[pl.ds(v * 16, 16)]
        e0 = jnp.sum(lax.fori_loop(0, NW, sb, jnp.zeros((16,), jnp.int32)))
        eh = jnp.where(cid == 0, e0, E - e0)
        oh = jnp.where(cid == 0, 0, _rup_t(e0))
        nb2 = _rup_t(eh) // (2 * B)

        half = lanes >> 3
        cols = lanes & 7
        himask = jnp.full((16,), -65536, jnp.int32)

        def process(dref, rref):
            @plsc.parallel_loop(0, B // 16, unroll=1)
            def _(i):
                dv16 = dref[pl.ds(i * 16, 16)]
                for u in range(8):
                    w = plsc.load_gather(rref, [half + (i * 16 + 2 * u), cols])
                    flo = plsc.bitcast(w << 16, jnp.float32)
                    fhi = plsc.bitcast(w & himask, jnp.float32)
                    dstv = dv16.at[half + 2 * u].get(mode="promise_in_bounds")
                    plsc.addupdate_scatter(acc, [dstv + cols], flo)
                    plsc.addupdate_scatter(acc, [dstv + cols + 8], fhi)

        def one_pass(p, _):
            chunk = p * 16 + sid
            xview = x3_hbm.at[pl.ds(p * (16 * N) + sid, nv16)]

            @plsc.parallel_loop(0, NH, unroll=8)
            def _(i):
                acc[pl.ds(i * 16, 16)] = zero
            acc[pl.ds(16 * NH, 16)] = zero

            pltpu.sync_copy(srcp_hbm.at[pl.ds(oh, B)], idxb.at[0])
            pltpu.async_copy(xview.at[idxb.at[0]], rows.at[0], g0)
            pltpu.async_copy(srcp_hbm.at[pl.ds(oh + B, B)], idxb.at[1], s1)
            pltpu.async_copy(dstp_hbm.at[pl.ds(oh, B)], dstb.at[0], d0)
            pltpu.async_copy(dstp_hbm.at[pl.ds(oh + B, B)], dstb.at[1], d1)

            def bb(k, _):
                off = oh + 2 * k * B
                more = k < nb2 - 1
                pltpu.make_async_copy(
                    srcp_hbm.at[pl.ds(off + B, B)], idxb.at[1], s1).wait()
                pltpu.async_copy(xview.at[idxb.at[1]], rows.at[1], g1)
                pltpu.make_async_copy(
                    xview.at[idxb.at[0]], rows.at[0], g0).wait()

                @pl.when(more)
                def _():
                    pltpu.async_copy(
                        srcp_hbm.at[pl.ds(off + 2 * B, B)], idxb.at[0], s0)
                pltpu.make_async_copy(
                    dstp_hbm.at[pl.ds(off, B)], dstb.at[0], d0).wait()
                process(dstb.at[0], rows.at[0])

                @pl.when(more)
                def _():
                    pltpu.async_copy(
                        dstp_hbm.at[pl.ds(off + 2 * B, B)], dstb.at[0], d0)
                    pltpu.make_async_copy(
                        srcp_hbm.at[pl.ds(off + 2 * B, B)], idxb.at[0], s0).wait()
                    pltpu.async_copy(xview.at[idxb.at[0]], rows.at[0], g0)
                pltpu.make_async_copy(
                    xview.at[idxb.at[1]], rows.at[1], g1).wait()

                @pl.when(more)
                def _():
                    pltpu.async_copy(
                        srcp_hbm.at[pl.ds(off + 3 * B, B)], idxb.at[1], s1)
                pltpu.make_async_copy(
                    dstp_hbm.at[pl.ds(off + B, B)], dstb.at[1], d1).wait()
                process(dstb.at[1], rows.at[1])

                @pl.when(more)
                def _():
                    pltpu.async_copy(
                        dstp_hbm.at[pl.ds(off + 3 * B, B)], dstb.at[1], d1)
                return 0
            lax.fori_loop(0, nb2, bb, 0)

            pltpu.sync_copy(acc.at[pl.ds(0, 16 * NH)],
                            out_hbm.at[chunk].at[pl.ds(cid * (16 * NH), 16 * NH)])
            return 0
        lax.fori_loop(0, npass, one_pass, 0)
    return agg


_agg1 = _make_agg(1)
_agg2 = _make_agg(2)


def _tc0_body(degp_ref, dinv_ref):
    deg = jnp.sum(degp_ref[...], axis=0) + 1.0
    dinv_ref[...] = lax.rsqrt(deg)[:, None]


def _pack16(a, b):
    ai = lax.bitcast_convert_type(a, jnp.int32) + 0x8000
    bi = lax.bitcast_convert_type(b, jnp.int32) + 0x8000
    return ((ai >> 16) & 0xFFFF) | (bi & -65536)


def _tc1_body(dinv_ref, x_ref, w1_ref, xws_ref, xpk_ref):
    xw = jnp.dot(x_ref[...], w1_ref[...], preferred_element_type=jnp.float32)
    xws = xw * dinv_ref[...]
    xws_ref[...] = xws
    xpk_ref[...] = _pack16(xws[:, :128], xws[:, 128:])


def _tc2_body(agg_ref, xws_ref, dinv_ref, b1_ref, w2_ref, o_ref, obf_ref):
    h = jnp.maximum(dinv_ref[...] * (agg_ref[...] + xws_ref[...]) + b1_ref[...], 0.0)
    xw2 = jnp.dot(h, w2_ref[...], preferred_element_type=jnp.float32)
    xws2 = xw2 * dinv_ref[...]
    o_ref[0] = xws2[:, :D]
    o_ref[1] = xws2[:, D:]
    obf_ref[0] = _pack16(xws2[:, 0:128], xws2[:, 256:384])
    obf_ref[1] = _pack16(xws2[:, 128:256], xws2[:, 384:512])


def _tc3_body(agg2_ref, x2_ref, dinv_ref, b2_ref, out_ref):
    i = pl.program_id(0)
    xws2 = jnp.concatenate([x2_ref[0], x2_ref[1]], axis=1)
    h2 = jnp.maximum(dinv_ref[...] * (agg2_ref[...] + xws2) + b2_ref[...], 0.0)
    part = jnp.sum(h2, axis=0, keepdims=True)

    @pl.when(i == 0)
    def _():
        out_ref[...] = part

    @pl.when(i > 0)
    def _():
        out_ref[...] = out_ref[...] + part

    @pl.when(i == N // RB - 1)
    def _():
        out_ref[...] = out_ref[...] * (1.0 / N)


_tc0 = pl.pallas_call(
    _tc0_body,
    in_specs=[pl.BlockSpec((NW, N), lambda: (0, 0))],
    out_specs=pl.BlockSpec((N, 1), lambda: (0, 0)),
    out_shape=jax.ShapeDtypeStruct((N, 1), jnp.float32),
)

_tc1 = pl.pallas_call(
    _tc1_body,
    grid=(N // RB,),
    in_specs=[
        pl.BlockSpec((RB, 1), lambda i: (i, 0)),
        pl.BlockSpec((RB, D), lambda i: (i, 0)),
        pl.BlockSpec((D, D), lambda i: (0, 0)),
    ],
    out_specs=[
        pl.BlockSpec((RB, D), lambda i: (i, 0)),
        pl.BlockSpec((RB, 128), lambda i: (i, 0)),
    ],
    out_shape=[
        jax.ShapeDtypeStruct((N, D), jnp.float32),
        jax.ShapeDtypeStruct((N, 128), jnp.int32),
    ],
)

_tc2 = pl.pallas_call(
    _tc2_body,
    grid=(N // RB,),
    in_specs=[
        pl.BlockSpec((RB, D), lambda i: (i, 0)),
        pl.BlockSpec((RB, D), lambda i: (i, 0)),
        pl.BlockSpec((RB, 1), lambda i: (i, 0)),
        pl.BlockSpec((1, D), lambda i: (0, 0)),
        pl.BlockSpec((D, 2 * D), lambda i: (0, 0)),
    ],
    out_specs=[
        pl.BlockSpec((2, RB, D), lambda i: (0, i, 0)),
        pl.BlockSpec((2, RB, 128), lambda i: (0, i, 0)),
    ],
    out_shape=[
        jax.ShapeDtypeStruct((2, N, D), jnp.float32),
        jax.ShapeDtypeStruct((2, N, 128), jnp.int32),
    ],
)

_tc3 = pl.pallas_call(
    _tc3_body,
    grid=(N // RB,),
    in_specs=[
        pl.BlockSpec((RB, 2 * D), lambda i: (i, 0)),
        pl.BlockSpec((2, RB, D), lambda i: (0, i, 0)),
        pl.BlockSpec((RB, 1), lambda i: (i, 0)),
        pl.BlockSpec((1, 2 * D), lambda i: (0, 0)),
    ],
    out_specs=pl.BlockSpec((1, 2 * D), lambda i: (0, 0)),
    out_shape=jax.ShapeDtypeStruct((1, 2 * D), jnp.float32),
)


def kernel(x, edge_index, W1, b1, W2, b2):
    src = edge_index[0].astype(jnp.int32)
    dst = edge_index[1].astype(jnp.int32)
    degp, cnt = _deg_kernel(dst)
    srcp, dstp = _part_kernel(src, dst, cnt)

    dinv = _tc0(degp)
    xws1, xpk1 = _tc1(dinv, x, W1)
    agg1 = _agg1(xpk1.reshape(N * 16, 8), srcp, dstp, cnt)
    agg1t = (agg1.reshape(16, N, 2, 8).transpose(1, 2, 0, 3).reshape(N, D))

    xws2q, xpk2 = _tc2(agg1t, xws1, dinv, b1.reshape(1, D), W2)
    agg2 = _agg2(xpk2.reshape(2 * N * 16, 8), srcp, dstp, cnt)
    agg2t = (agg2.reshape(32, N, 2, 8).transpose(1, 2, 0, 3).reshape(N, 2 * D))

    out = _tc3(agg2t, xws2q, dinv, b2.reshape(1, 2 * D))
    return out.reshape(2 * D)


# speedup vs baseline: 1.9374x; 1.9374x over previous
import functools

import jax
import jax.numpy as jnp
from jax import lax
from jax.experimental import pallas as pl
from jax.experimental.pallas import tpu as pltpu
from jax.experimental.pallas import tpu_sc as plsc

N = 10000
D = 256
E = 160000
NW = 32
EPW = E // NW
B = 2000
RB = 1000

_mesh = lambda: plsc.VectorSubcoreMesh(
    core_axis_name="c", subcore_axis_name="s", num_cores=2, num_subcores=16)
_sc_params = pltpu.CompilerParams(
    needs_layout_passes=False, use_tc_tiling_on_sc=False)


@functools.partial(
    pl.kernel,
    out_type=(
        jax.ShapeDtypeStruct((NW, N), jnp.float32),
        jax.ShapeDtypeStruct((E,), jnp.int32),
        jax.ShapeDtypeStruct((E,), jnp.int32),
    ),
    mesh=_mesh(),
    scratch_types=[
        pltpu.VMEM((EPW + 16,), jnp.int32),
        pltpu.VMEM((EPW + 16,), jnp.int32),
        pltpu.VMEM((EPW + 16,), jnp.int32),
        pltpu.VMEM((8 * N,), jnp.float32),
    ],
    compiler_params=_sc_params,
)
def _deg_kernel(src_hbm, dst_hbm, degp_hbm, src32_hbm, dst8_hbm,
                srcb, dstb, d8b, acc):
    wid = lax.axis_index("s") * 2 + lax.axis_index("c")
    base = wid * EPW
    pltpu.sync_copy(src_hbm.at[pl.ds(base, EPW)], srcb.at[pl.ds(0, EPW)])
    pltpu.sync_copy(dst_hbm.at[pl.ds(base, EPW)], dstb.at[pl.ds(0, EPW)])

    lanes = lax.iota(jnp.int32, 16)
    offs = (lanes & 7) * N
    mlo = lanes < 8
    mhi = lanes >= 8
    ones = jnp.ones((16,), jnp.float32)
    zero = jnp.zeros((16,), jnp.float32)

    @plsc.parallel_loop(0, (8 * N) // 16, unroll=8)
    def _(i):
        acc[pl.ds(i * 16, 16)] = zero

    nfull = EPW // 16

    @plsc.parallel_loop(0, nfull, unroll=8)
    def _(i):
        j = i * 16
        sv = srcb[pl.ds(j, 16)]
        srcb[pl.ds(j, 16)] = sv * 32
        dv = dstb[pl.ds(j, 16)]
        d8b[pl.ds(j, 16)] = dv * 8
        addr = dv + offs
        plsc.addupdate_scatter(acc, [addr], ones, mask=mlo)
        plsc.addupdate_scatter(acc, [addr], ones, mask=mhi)

    j = nfull * 16
    sv = srcb[pl.ds(j, 16)]
    srcb[pl.ds(j, 16)] = sv * 32
    dv = dstb[pl.ds(j, 16)]
    d8b[pl.ds(j, 16)] = dv * 8
    plsc.addupdate_scatter(acc, [dv + offs], ones, mask=mlo)

    @plsc.parallel_loop(0, N // 16, unroll=4)
    def _(i):
        s = acc[pl.ds(i * 16, 16)]
        for k in range(1, 8):
            s = s + acc[pl.ds(k * N + i * 16, 16)]
        acc[pl.ds(i * 16, 16)] = s

    pltpu.sync_copy(srcb.at[pl.ds(0, EPW)], src32_hbm.at[pl.ds(base, EPW)])
    pltpu.sync_copy(d8b.at[pl.ds(0, EPW)], dst8_hbm.at[pl.ds(base, EPW)])
    pltpu.sync_copy(acc.at[pl.ds(0, N)], degp_hbm.at[wid])


def _make_agg(K):
    npass = K // NW

    nb = E // B
    nb2 = nb // 2

    @functools.partial(
        pl.kernel,
        out_type=jax.ShapeDtypeStruct((K, 8 * N), jnp.float32),
        mesh=_mesh(),
        scratch_types=[
            pltpu.VMEM((2, B), jnp.int32),
            pltpu.VMEM((2, B), jnp.int32),
            pltpu.VMEM((2, B, 8), jnp.float32),
            pltpu.VMEM((8 * N,), jnp.float32),
            pltpu.SemaphoreType.DMA,
            pltpu.SemaphoreType.DMA,
            pltpu.SemaphoreType.DMA,
            pltpu.SemaphoreType.DMA,
            pltpu.SemaphoreType.DMA,
            pltpu.SemaphoreType.DMA,
        ],
        compiler_params=_sc_params,
    )
    def agg(xflat_hbm, srck_hbm, dst8_hbm, out_hbm, idxb, dstb, rows, acc,
            s0, s1, g0, g1, d0, d1):
        wid = lax.axis_index("s") * 2 + lax.axis_index("c")
        lanes = lax.iota(jnp.int32, 16)
        feat = lanes & 7
        half = lanes >> 3
        mlo = lanes < 8
        mhi = lanes >= 8
        zero = jnp.zeros((16,), jnp.float32)

        def process(dref, rref):
            @plsc.parallel_loop(0, B // 2, unroll=8)
            def _(k):
                ev = half + 2 * k
                dstv = plsc.load_gather(dref, [ev])
                addr = dstv + feat
                row = plsc.load_gather(rref, [ev, feat])
                plsc.addupdate_scatter(acc, [addr], row)

        nview = N * K - K + 1
        for p in range(npass):
            chunk = wid + NW * p
            xview = xflat_hbm.at[pl.ds(chunk, nview)]

            @plsc.parallel_loop(0, (8 * N) // 16, unroll=8)
            def _(i):
                acc[pl.ds(i * 16, 16)] = zero

            pltpu.sync_copy(srck_hbm.at[pl.ds(0, B)], idxb.at[0])
            pltpu.async_copy(xview.at[idxb.at[0]], rows.at[0], g0)
            pltpu.async_copy(srck_hbm.at[pl.ds(B, B)], idxb.at[1], s1)
            pltpu.async_copy(dst8_hbm.at[pl.ds(0, B)], dstb.at[0], d0)
            pltpu.async_copy(dst8_hbm.at[pl.ds(B, B)], dstb.at[1], d1)

            def bb(k, _):
                off = 2 * k * B
                more = k < nb2 - 1
                pltpu.make_async_copy(
                    srck_hbm.at[pl.ds(off + B, B)], idxb.at[1], s1).wait()
                pltpu.async_copy(xview.at[idxb.at[1]], rows.at[1], g1)
                pltpu.make_async_copy(
                    xview.at[idxb.at[0]], rows.at[0], g0).wait()

                @pl.when(more)
                def _():
                    pltpu.async_copy(
                        srck_hbm.at[pl.ds(off + 2 * B, B)], idxb.at[0], s0)
                pltpu.make_async_copy(
                    dst8_hbm.at[pl.ds(off, B)], dstb.at[0], d0).wait()
                process(dstb.at[0], rows.at[0])

                @pl.when(more)
                def _():
                    pltpu.async_copy(
                        dst8_hbm.at[pl.ds(off + 2 * B, B)], dstb.at[0], d0)
                    pltpu.make_async_copy(
                        srck_hbm.at[pl.ds(off + 2 * B, B)], idxb.at[0], s0).wait()
                    pltpu.async_copy(xview.at[idxb.at[0]], rows.at[0], g0)
                pltpu.make_async_copy(
                    xview.at[idxb.at[1]], rows.at[1], g1).wait()

                @pl.when(more)
                def _():
                    pltpu.async_copy(
                        srck_hbm.at[pl.ds(off + 3 * B, B)], idxb.at[1], s1)
                pltpu.make_async_copy(
                    dst8_hbm.at[pl.ds(off + B, B)], dstb.at[1], d1).wait()
                process(dstb.at[1], rows.at[1])

                @pl.when(more)
                def _():
                    pltpu.async_copy(
                        dst8_hbm.at[pl.ds(off + 3 * B, B)], dstb.at[1], d1)
                return 0
            lax.fori_loop(0, nb2, bb, 0)

            pltpu.sync_copy(acc, out_hbm.at[chunk])
    return agg


_agg32 = _make_agg(32)
_agg64 = _make_agg(64)


def _tc0_body(degp_ref, dinv_ref):
    deg = jnp.sum(degp_ref[...], axis=0) + 1.0
    dinv_ref[...] = lax.rsqrt(deg)[:, None]


def _tc1_body(dinv_ref, x_ref, w1_ref, xws_ref):
    xw = jnp.dot(x_ref[...], w1_ref[...], preferred_element_type=jnp.float32)
    xws_ref[...] = xw * dinv_ref[...]


def _tc2_body(agg_ref, xws_ref, dinv_ref, b1_ref, w2_ref, xws2_ref):
    h = jnp.maximum(dinv_ref[...] * (agg_ref[...] + xws_ref[...]) + b1_ref[...], 0.0)
    xw2 = jnp.dot(h, w2_ref[...], preferred_element_type=jnp.float32)
    xws2_ref[...] = xw2 * dinv_ref[...]


def _tc3_body(agg2_ref, xws2_ref, dinv_ref, b2_ref, out_ref):
    i = pl.program_id(0)
    h2 = jnp.maximum(dinv_ref[...] * (agg2_ref[...] + xws2_ref[...]) + b2_ref[...], 0.0)
    part = jnp.sum(h2, axis=0, keepdims=True)

    @pl.when(i == 0)
    def _():
        out_ref[...] = part

    @pl.when(i > 0)
    def _():
        out_ref[...] = out_ref[...] + part

    @pl.when(i == N // RB - 1)
    def _():
        out_ref[...] = out_ref[...] * (1.0 / N)


_tc0 = pl.pallas_call(
    _tc0_body,
    in_specs=[pl.BlockSpec((NW, N), lambda: (0, 0))],
    out_specs=pl.BlockSpec((N, 1), lambda: (0, 0)),
    out_shape=jax.ShapeDtypeStruct((N, 1), jnp.float32),
)

_tc1 = pl.pallas_call(
    _tc1_body,
    grid=(N // RB,),
    in_specs=[
        pl.BlockSpec((RB, 1), lambda i: (i, 0)),
        pl.BlockSpec((RB, D), lambda i: (i, 0)),
        pl.BlockSpec((D, D), lambda i: (0, 0)),
    ],
    out_specs=pl.BlockSpec((RB, D), lambda i: (i, 0)),
    out_shape=jax.ShapeDtypeStruct((N, D), jnp.float32),
)

_tc2 = pl.pallas_call(
    _tc2_body,
    grid=(N // RB,),
    in_specs=[
        pl.BlockSpec((RB, D), lambda i: (i, 0)),
        pl.BlockSpec((RB, D), lambda i: (i, 0)),
        pl.BlockSpec((RB, 1), lambda i: (i, 0)),
        pl.BlockSpec((1, D), lambda i: (0, 0)),
        pl.BlockSpec((D, 2 * D), lambda i: (0, 0)),
    ],
    out_specs=pl.BlockSpec((RB, 2 * D), lambda i: (i, 0)),
    out_shape=jax.ShapeDtypeStruct((N, 2 * D), jnp.float32),
)

_tc3 = pl.pallas_call(
    _tc3_body,
    grid=(N // RB,),
    in_specs=[
        pl.BlockSpec((RB, 2 * D), lambda i: (i, 0)),
        pl.BlockSpec((RB, 2 * D), lambda i: (i, 0)),
        pl.BlockSpec((RB, 1), lambda i: (i, 0)),
        pl.BlockSpec((1, 2 * D), lambda i: (0, 0)),
    ],
    out_specs=pl.BlockSpec((1, 2 * D), lambda i: (0, 0)),
    out_shape=jax.ShapeDtypeStruct((1, 2 * D), jnp.float32),
)


def kernel(x, edge_index, W1, b1, W2, b2):
    src = edge_index[0].astype(jnp.int32)
    dst = edge_index[1].astype(jnp.int32)
    degp, src32, dst8 = _deg_kernel(src, dst)

    dinv = _tc0(degp)
    xws1 = _tc1(dinv, x, W1)
    agg1 = _agg32(xws1.reshape(N * 32, 8), src32, dst8)
    agg1t = agg1.reshape(32, N, 8).transpose(1, 0, 2).reshape(N, D)

    xws2 = _tc2(agg1t, xws1, dinv, b1.reshape(1, D), W2)
    agg2 = _agg64(xws2.reshape(N * 64, 8), src32 + src32, dst8)
    agg2t = agg2.reshape(64, N, 8).transpose(1, 0, 2).reshape(N, 2 * D)

    out = _tc3(agg2t, xws2, dinv, b2.reshape(1, 2 * D))
    return out.reshape(2 * D)
